# Initial kernel scaffold; baseline (speedup 1.0000x reference)
import jax
import jax.numpy as jnp
from jax.experimental import pallas as pl


def kernel(x, edge_index, index, W0, b0, W1, b1, g1, be1, W2, b2, g2, be2, W3, b3, g3, be3, Wf, bf):
    raise NotImplementedError("write your pallas kernel here")


# speedup vs baseline: 10.8961x; 10.8961x over previous
import functools

import jax
import jax.numpy as jnp
from jax import lax
from jax.experimental import pallas as pl
from jax.experimental.pallas import tpu as pltpu
from jax.experimental.pallas import tpu_sc as plsc

_N = 10000
_E = 160000
_IN_C = 256
_HID = 512
_OUT_C = 10
_G = 64

_NP = 10240
_EP = 163840
_NB = 40
_B = 128
_RB = 400
_NBLK = _N // _RB


def _sc_mesh():
    return plsc.VectorSubcoreMesh(core_axis_name="c", subcore_axis_name="s")


def _fill_const(ref, rows, val):
    v = jnp.full((16,), val, jnp.float32)

    def body(i, _):
        for k in range(8):
            ref[i, pl.ds(k * 16, 16)] = v
        return 0

    lax.fori_loop(0, rows, body, 0)


def _sc_degree(dstp):
    @functools.partial(
        pl.kernel,
        out_type=jax.ShapeDtypeStruct((2, _NP), jnp.float32),
        mesh=_sc_mesh(),
        scratch_types=[
            pltpu.VMEM((_NB, _B), jnp.int32),
            pltpu.VMEM((_B,), jnp.float32),
            pltpu.VMEM((640,), jnp.float32),
            pltpu.VMEM_SHARED((_NP,), jnp.float32),
        ],
    )
    def k(dst_hbm, out_hbm, idx_v, ones_v, zero_v, deg_sh):
        c = lax.axis_index("c")
        s = lax.axis_index("s")
        wid = c * 16 + s
        pltpu.sync_copy(dst_hbm.at[wid], idx_v)
        one16 = jnp.full((16,), 1.0, jnp.float32)
        z16 = jnp.zeros((16,), jnp.float32)
        for k8 in range(8):
            ones_v[pl.ds(k8 * 16, 16)] = one16

        def zbody(i, _):
            zero_v[pl.ds(i * 16, 16)] = z16
            return 0

        lax.fori_loop(0, 40, zbody, 0)
        pltpu.sync_copy(zero_v, deg_sh.at[pl.ds(s * 640, 640)])
        plsc.subcore_barrier()

        def body(j, _):
            pltpu.sync_copy(ones_v, deg_sh.at[idx_v.at[j]], add=True)
            return 0

        lax.fori_loop(0, _NB, body, 0)
        plsc.subcore_barrier()
        pltpu.sync_copy(deg_sh.at[pl.ds(s * 640, 640)],
                        out_hbm.at[c, pl.ds(s * 640, 640)])

    return k(dstp)


def _sc_prop(uf, srcp, dstp, nch):
    nchc = nch // 2

    @functools.partial(
        pl.kernel,
        out_type=jax.ShapeDtypeStruct((nch, _NP, 128), jnp.float32),
        mesh=_sc_mesh(),
        scratch_types=[
            pltpu.VMEM((_NB, _B), jnp.int32),
            pltpu.VMEM((_NB, _B), jnp.int32),
            pltpu.VMEM((_B,), jnp.int32),
            pltpu.VMEM((_B,), jnp.int32),
            pltpu.VMEM((_B, 128), jnp.float32),
            pltpu.VMEM((_B, 128), jnp.float32),
            pltpu.VMEM_SHARED((_NP, 128), jnp.float32),
            pltpu.SemaphoreType.DMA,
            pltpu.SemaphoreType.DMA,
        ],
    )
    def k(u_hbm, src_hbm, dst_hbm, out_hbm,
          idxs_v, idxd_v, idxg0, idxg1, rows0, rows1, z_sh,
          sem0, sem1):
        c = lax.axis_index("c")
        s = lax.axis_index("s")

        for chl in range(nchc):
            chunk = c * nchc + chl
            off = chunk * _N
            _fill_const(rows0, _B, 0.0)

            def zbody(i, _):
                pltpu.sync_copy(rows0, z_sh.at[pl.ds(s * 640 + i * 128, 128)])
                return 0

            lax.fori_loop(0, 5, zbody, 0)
            plsc.subcore_barrier()

            for half in range(2):
                row = s * 2 + half
                pltpu.sync_copy(src_hbm.at[row], idxs_v)
                pltpu.sync_copy(dst_hbm.at[row], idxd_v)

                def body(i, _):
                    j0 = 2 * i
                    j1 = 2 * i + 1
                    for k8 in range(8):
                        idxg0[pl.ds(k8 * 16, 16)] = (
                            idxs_v[j0, pl.ds(k8 * 16, 16)] + off)
                    for k8 in range(8):
                        idxg1[pl.ds(k8 * 16, 16)] = (
                            idxs_v[j1, pl.ds(k8 * 16, 16)] + off)
                    cp0 = pltpu.async_copy(u_hbm.at[idxg0], rows0, sem0)
                    cp1 = pltpu.async_copy(u_hbm.at[idxg1], rows1, sem1)
                    cp0.wait()
                    pltpu.sync_copy(rows0, z_sh.at[idxd_v.at[j0]], add=True)
                    cp1.wait()
                    pltpu.sync_copy(rows1, z_sh.at[idxd_v.at[j1]], add=True)
                    return 0

                lax.fori_loop(0, _NB // 2, body, 0)
            plsc.subcore_barrier()
            pltpu.sync_copy(z_sh.at[pl.ds(s * 640, 640)],
                            out_hbm.at[chunk, pl.ds(s * 640, 640)])
            plsc.subcore_barrier()

    return k(uf, srcp, dstp)[:, :_N]


def _tc_scale0(deg2, x):
    def body(deg_ref, x_ref, dinv_ref, u_ref):
        d = deg_ref[0] + deg_ref[1] + 1.0
        dv = lax.rsqrt(d)
        dinv_ref[...] = dv
        for ch in range(2):
            u_ref[ch] = x_ref[:, ch * 128:(ch + 1) * 128] * dv

    return pl.pallas_call(
        body,
        grid=(_NBLK,),
        in_specs=[
            pl.BlockSpec((2, _RB, 1), lambda i: (0, i, 0)),
            pl.BlockSpec((_RB, _IN_C), lambda i: (i, 0)),
        ],
        out_specs=[
            pl.BlockSpec((_RB, 1), lambda i: (i, 0)),
            pl.BlockSpec((2, _RB, 128), lambda i: (0, i, 0)),
        ],
        out_shape=[
            jax.ShapeDtypeStruct((_N, 1), jnp.float32),
            jax.ShapeDtypeStruct((2, _N, 128), jnp.float32),
        ],
    )(deg2, x)


def _tc_combine(z, u, dinv, W, b, r):
    nch = z.shape[0]
    cin = nch * 128
    has_r = r is not None

    def body(*refs):
        if has_r:
            (z_ref, u_ref, dinv_ref, w_ref, b_ref, r_ref,
             h_ref, st_ref) = refs
        else:
            z_ref, u_ref, dinv_ref, w_ref, b_ref, h_ref, st_ref = refs
        i = pl.program_id(0)
        v = jnp.concatenate([z_ref[ch] + u_ref[ch] for ch in range(nch)],
                            axis=1)
        h = jnp.dot(v, w_ref[...], preferred_element_type=jnp.float32)
        h = h * dinv_ref[...] + b_ref[...]
        if has_r:
            h = h + r_ref[...]
        h_ref[...] = h

        @pl.when(i == 0)
        def _():
            st_ref[...] = jnp.zeros_like(st_ref)

        st_ref[...] += jnp.stack(
            [jnp.sum(h, axis=0), jnp.sum(h * h, axis=0)])

    in_specs = [
        pl.BlockSpec((nch, _RB, 128), lambda i: (0, i, 0)),
        pl.BlockSpec((nch, _RB, 128), lambda i: (0, i, 0)),
        pl.BlockSpec((_RB, 1), lambda i: (i, 0)),
        pl.BlockSpec((cin, _HID), lambda i: (0, 0)),
        pl.BlockSpec((1, _HID), lambda i: (0, 0)),
    ]
    args = [z, u, dinv, W, b.reshape(1, _HID)]
    if has_r:
        in_specs.append(pl.BlockSpec((_RB, _HID), lambda i: (i, 0)))
        args.append(r)
    return pl.pallas_call(
        body,
        grid=(_NBLK,),
        in_specs=in_specs,
        out_specs=[
            pl.BlockSpec((_RB, _HID), lambda i: (i, 0)),
            pl.BlockSpec((2, _HID), lambda i: (0, 0)),
        ],
        out_shape=[
            jax.ShapeDtypeStruct((_N, _HID), jnp.float32),
            jax.ShapeDtypeStruct((2, _HID), jnp.float32),
        ],
    )(*args)


def _tc_bn_scale(h, st, g, be, dinv):
    def body(h_ref, st_ref, g_ref, be_ref, dinv_ref, u_ref):
        m = st_ref[0:1] / float(_N)
        var = st_ref[1:2] / float(_N) - m * m
        sc = g_ref[...] / jnp.sqrt(var + 1e-5)
        sh = be_ref[...] - m * sc
        t = jnp.maximum(h_ref[...] * sc + sh, 0.0) * dinv_ref[...]
        for ch in range(4):
            u_ref[ch] = t[:, ch * 128:(ch + 1) * 128]

    return pl.pallas_call(
        body,
        grid=(_NBLK,),
        in_specs=[
            pl.BlockSpec((_RB, _HID), lambda i: (i, 0)),
            pl.BlockSpec((2, _HID), lambda i: (0, 0)),
            pl.BlockSpec((1, _HID), lambda i: (0, 0)),
            pl.BlockSpec((1, _HID), lambda i: (0, 0)),
            pl.BlockSpec((_RB, 1), lambda i: (i, 0)),
        ],
        out_specs=pl.BlockSpec((4, _RB, 128), lambda i: (0, i, 0)),
        out_shape=jax.ShapeDtypeStruct((4, _N, 128), jnp.float32),
    )(h, st, g.reshape(1, _HID), be.reshape(1, _HID), dinv)


def _tc_final(z, u, dinv, W, b, r, index, Wf, bf):
    def body(z_ref, u_ref, dinv_ref, w_ref, b_ref, r_ref, idx_ref,
             wf_ref, bf_ref, o_ref, pooled):
        i = pl.program_id(0)
        v = jnp.concatenate([z_ref[ch] + u_ref[ch] for ch in range(4)],
                            axis=1)
        h = jnp.dot(v, w_ref[...], preferred_element_type=jnp.float32)
        h = h * dinv_ref[...] + b_ref[...] + r_ref[...]
        seg = (idx_ref[...] == jax.lax.broadcasted_iota(
            jnp.int32, (1, _G), 1)).astype(jnp.float32)

        @pl.when(i == 0)
        def _():
            pooled[...] = jnp.zeros_like(pooled)

        pooled[...] += lax.dot_general(
            seg, h, (((0,), (0,)), ((), ())),
            preferred_element_type=jnp.float32)

        @pl.when(i == _NBLK - 1)
        def _():
            o_ref[...] = jnp.dot(
                pooled[...], wf_ref[...],
                preferred_element_type=jnp.float32) + bf_ref[...]

    return pl.pallas_call(
        body,
        grid=(_NBLK,),
        in_specs=[
            pl.BlockSpec((4, _RB, 128), lambda i: (0, i, 0)),
            pl.BlockSpec((4, _RB, 128), lambda i: (0, i, 0)),
            pl.BlockSpec((_RB, 1), lambda i: (i, 0)),
            pl.BlockSpec((_HID, _HID), lambda i: (0, 0)),
            pl.BlockSpec((1, _HID), lambda i: (0, 0)),
            pl.BlockSpec((_RB, _HID), lambda i: (i, 0)),
            pl.BlockSpec((_RB, 1), lambda i: (i, 0)),
            pl.BlockSpec((_HID, _OUT_C), lambda i: (0, 0)),
            pl.BlockSpec((1, _OUT_C), lambda i: (0, 0)),
        ],
        out_specs=pl.BlockSpec((_G, _OUT_C), lambda i: (0, 0)),
        out_shape=jax.ShapeDtypeStruct((_G, _OUT_C), jnp.float32),
        scratch_shapes=[pltpu.VMEM((_G, _HID), jnp.float32)],
    )(z, u, dinv, W, b.reshape(1, _HID), r, index.reshape(_N, 1),
      Wf, bf.reshape(1, _OUT_C))


def kernel(x, edge_index, index, W0, b0, W1, b1, g1, be1, W2, b2, g2, be2,
           W3, b3, g3, be3, Wf, bf):
    src = edge_index[0]
    dst = edge_index[1]
    pad = _EP - _E
    ar = jnp.arange(pad, dtype=jnp.int32)
    pad_src = (ar * 7919) % _N
    pad_dst = _N + (ar % (_NP - _N))
    srcp = jnp.concatenate([src, pad_src]).reshape(32, _NB, _B)
    dstp = jnp.concatenate([dst, pad_dst]).reshape(32, _NB, _B)

    degp = _sc_degree(dstp)
    deg2 = degp[:, :_N].reshape(2, _N, 1)
    dinv, u0 = _tc_scale0(deg2, x)

    z0 = _sc_prop(u0.reshape(2 * _N, 128), srcp, dstp, 2)
    h, st = _tc_combine(z0, u0, dinv, W0, b0, None)

    for (W, b, g, be) in ((W1, b1, g1, be1), (W2, b2, g2, be2)):
        u = _tc_bn_scale(h, st, g, be, dinv)
        z = _sc_prop(u.reshape(4 * _N, 128), srcp, dstp, 4)
        h, st = _tc_combine(z, u, dinv, W, b, h)

    u3 = _tc_bn_scale(h, st, g3, be3, dinv)
    z3 = _sc_prop(u3.reshape(4 * _N, 128), srcp, dstp, 4)
    return _tc_final(z3, u3, dinv, W3, b3, h, index, Wf, bf)


# speedup vs baseline: 13.5679x; 1.2452x over previous
import functools

import jax
import jax.numpy as jnp
from jax import lax
from jax.experimental import pallas as pl
from jax.experimental.pallas import tpu as pltpu
from jax.experimental.pallas import tpu_sc as plsc

_N = 10000
_E = 160000
_IN_C = 256
_HID = 512
_OUT_C = 10
_G = 64

_NP = 10240
_EP = 163840
_NB = 40
_B = 128
_RB = 400
_NBLK = _N // _RB


def _sc_mesh():
    return plsc.VectorSubcoreMesh(core_axis_name="c", subcore_axis_name="s")


def _fill_const(ref, rows, val):
    v = jnp.full((16,), val, jnp.float32)

    def body(i, _):
        for k in range(8):
            ref[i, pl.ds(k * 16, 16)] = v
        return 0

    lax.fori_loop(0, rows, body, 0)


def _sc_degree(dstp):
    @functools.partial(
        pl.kernel,
        out_type=jax.ShapeDtypeStruct((2, _NP), jnp.float32),
        mesh=_sc_mesh(),
        scratch_types=[
            pltpu.VMEM((_NB, _B), jnp.int32),
            pltpu.VMEM((_B,), jnp.float32),
            pltpu.VMEM((640,), jnp.float32),
            pltpu.VMEM_SHARED((_NP,), jnp.float32),
        ],
    )
    def k(dst_hbm, out_hbm, idx_v, ones_v, zero_v, deg_sh):
        c = lax.axis_index("c")
        s = lax.axis_index("s")
        wid = c * 16 + s
        pltpu.sync_copy(dst_hbm.at[wid], idx_v)
        one16 = jnp.full((16,), 1.0, jnp.float32)
        z16 = jnp.zeros((16,), jnp.float32)
        for k8 in range(8):
            ones_v[pl.ds(k8 * 16, 16)] = one16

        def zbody(i, _):
            zero_v[pl.ds(i * 16, 16)] = z16
            return 0

        lax.fori_loop(0, 40, zbody, 0)
        pltpu.sync_copy(zero_v, deg_sh.at[pl.ds(s * 640, 640)])
        plsc.subcore_barrier()

        def body(j, _):
            pltpu.sync_copy(ones_v, deg_sh.at[idx_v.at[j]], add=True)
            return 0

        lax.fori_loop(0, _NB, body, 0)
        plsc.subcore_barrier()
        pltpu.sync_copy(deg_sh.at[pl.ds(s * 640, 640)],
                        out_hbm.at[c, pl.ds(s * 640, 640)])

    return k(dstp)


_BS = 64
_NBS = 160


def _sc_prop(uf, src_off, dstp2, nch):
    nchc = nch // 2

    @functools.partial(
        pl.kernel,
        out_type=jax.ShapeDtypeStruct((nch, _NP, 128), jnp.float32),
        mesh=_sc_mesh(),
        scratch_types=[
            pltpu.VMEM((40, _BS), jnp.int32),
            pltpu.VMEM((40, _BS), jnp.int32),
            pltpu.VMEM((_BS, 128), jnp.float32),
            pltpu.VMEM((_BS, 128), jnp.float32),
            pltpu.VMEM((_BS, 128), jnp.float32),
            pltpu.VMEM((_BS, 128), jnp.float32),
            pltpu.SemaphoreType.DMA,
            pltpu.SemaphoreType.DMA,
            pltpu.SemaphoreType.DMA,
            pltpu.SemaphoreType.DMA,
            pltpu.SemaphoreType.DMA,
            pltpu.SemaphoreType.DMA,
            pltpu.SemaphoreType.DMA,
            pltpu.SemaphoreType.DMA,
            pltpu.VMEM_SHARED((_NP, 128), jnp.float32),
        ],
    )
    def k(u_hbm, srco_hbm, dst_hbm, out_hbm,
          idxs_v, idxd_v, rows0, rows1, rows2, rows3,
          sg0, sg1, sg2, sg3, ss0, ss1, ss2, ss3, z_sh):
        c = lax.axis_index("c")
        s = lax.axis_index("s")
        rows = (rows0, rows1, rows2, rows3)
        sg = (sg0, sg1, sg2, sg3)
        ss = (ss0, ss1, ss2, ss3)

        def gather(j, b):
            pltpu.async_copy(u_hbm.at[idxs_v.at[j]], rows[b], sg[b])

        def wait_g(b):
            pltpu.make_async_copy(
                u_hbm.at[idxs_v.at[0]], rows[b], sg[b]).wait()

        def scat(j, b):
            pltpu.async_copy(rows[b], z_sh.at[idxd_v.at[j]], ss[b],
                             add=True)

        def wait_s(b):
            pltpu.make_async_copy(
                rows[b], z_sh.at[idxd_v.at[0]], ss[b]).wait()

        for chl in range(nchc):
            chunk = c * nchc + chl
            _fill_const(rows0, _BS, 0.0)

            def zbody(i, _):
                pltpu.sync_copy(rows0, z_sh.at[pl.ds(s * 640 + i * 64, 64)])
                return 0

            lax.fori_loop(0, 10, zbody, 0)
            plsc.subcore_barrier()

            for q in range(4):
                pltpu.sync_copy(
                    srco_hbm.at[chunk, s, pl.ds(q * 40, 40)], idxs_v)
                pltpu.sync_copy(
                    dst_hbm.at[s, pl.ds(q * 40, 40)], idxd_v)
                for b in range(4):
                    gather(b, b)

                def body(i, _):
                    j = 4 * i
                    for p in range(2):
                        for b in (2 * p, 2 * p + 1):
                            wait_g(b)
                            scat(j + b, b)
                        for b in (2 * p, 2 * p + 1):
                            @pl.when(j + b + 4 < 40)
                            def _(b=b):
                                wait_s(b)
                                gather(j + b + 4, b)
                    return 0

                lax.fori_loop(0, 10, body, 0)
                for b in range(4):
                    wait_s(b)
            plsc.subcore_barrier()
            pltpu.sync_copy(z_sh.at[pl.ds(s * 640, 640)],
                            out_hbm.at[chunk, pl.ds(s * 640, 640)])
            plsc.subcore_barrier()

    return k(uf, src_off, dstp2)[:, :_N]


def _tc_scale0(deg2, x):
    def body(deg_ref, x_ref, dinv_ref, u_ref):
        d = deg_ref[0] + deg_ref[1] + 1.0
        dv = lax.rsqrt(d)
        dinv_ref[...] = dv
        for ch in range(2):
            u_ref[ch] = x_ref[:, ch * 128:(ch + 1) * 128] * dv

    return pl.pallas_call(
        body,
        grid=(_NBLK,),
        in_specs=[
            pl.BlockSpec((2, _RB, 1), lambda i: (0, i, 0)),
            pl.BlockSpec((_RB, _IN_C), lambda i: (i, 0)),
        ],
        out_specs=[
            pl.BlockSpec((_RB, 1), lambda i: (i, 0)),
            pl.BlockSpec((2, _RB, 128), lambda i: (0, i, 0)),
        ],
        out_shape=[
            jax.ShapeDtypeStruct((_N, 1), jnp.float32),
            jax.ShapeDtypeStruct((2, _N, 128), jnp.float32),
        ],
    )(deg2, x)


def _tc_combine(z, u, dinv, W, b, r):
    nch = z.shape[0]
    cin = nch * 128
    has_r = r is not None

    def body(*refs):
        if has_r:
            (z_ref, u_ref, dinv_ref, w_ref, b_ref, r_ref,
             h_ref, st_ref) = refs
        else:
            z_ref, u_ref, dinv_ref, w_ref, b_ref, h_ref, st_ref = refs
        i = pl.program_id(0)
        v = jnp.concatenate([z_ref[ch] + u_ref[ch] for ch in range(nch)],
                            axis=1)
        h = jnp.dot(v, w_ref[...], preferred_element_type=jnp.float32)
        h = h * dinv_ref[...] + b_ref[...]
        if has_r:
            h = h + r_ref[...]
        h_ref[...] = h

        @pl.when(i == 0)
        def _():
            st_ref[...] = jnp.zeros_like(st_ref)

        st_ref[...] += jnp.stack(
            [jnp.sum(h, axis=0), jnp.sum(h * h, axis=0)])

    in_specs = [
        pl.BlockSpec((nch, _RB, 128), lambda i: (0, i, 0)),
        pl.BlockSpec((nch, _RB, 128), lambda i: (0, i, 0)),
        pl.BlockSpec((_RB, 1), lambda i: (i, 0)),
        pl.BlockSpec((cin, _HID), lambda i: (0, 0)),
        pl.BlockSpec((1, _HID), lambda i: (0, 0)),
    ]
    args = [z, u, dinv, W, b.reshape(1, _HID)]
    if has_r:
        in_specs.append(pl.BlockSpec((_RB, _HID), lambda i: (i, 0)))
        args.append(r)
    return pl.pallas_call(
        body,
        grid=(_NBLK,),
        in_specs=in_specs,
        out_specs=[
            pl.BlockSpec((_RB, _HID), lambda i: (i, 0)),
            pl.BlockSpec((2, _HID), lambda i: (0, 0)),
        ],
        out_shape=[
            jax.ShapeDtypeStruct((_N, _HID), jnp.float32),
            jax.ShapeDtypeStruct((2, _HID), jnp.float32),
        ],
    )(*args)


def _tc_bn_scale(h, st, g, be, dinv):
    def body(h_ref, st_ref, g_ref, be_ref, dinv_ref, u_ref):
        m = st_ref[0:1] / float(_N)
        var = st_ref[1:2] / float(_N) - m * m
        sc = g_ref[...] / jnp.sqrt(var + 1e-5)
        sh = be_ref[...] - m * sc
        t = jnp.maximum(h_ref[...] * sc + sh, 0.0) * dinv_ref[...]
        for ch in range(4):
            u_ref[ch] = t[:, ch * 128:(ch + 1) * 128]

    return pl.pallas_call(
        body,
        grid=(_NBLK,),
        in_specs=[
            pl.BlockSpec((_RB, _HID), lambda i: (i, 0)),
            pl.BlockSpec((2, _HID), lambda i: (0, 0)),
            pl.BlockSpec((1, _HID), lambda i: (0, 0)),
            pl.BlockSpec((1, _HID), lambda i: (0, 0)),
            pl.BlockSpec((_RB, 1), lambda i: (i, 0)),
        ],
        out_specs=pl.BlockSpec((4, _RB, 128), lambda i: (0, i, 0)),
        out_shape=jax.ShapeDtypeStruct((4, _N, 128), jnp.float32),
    )(h, st, g.reshape(1, _HID), be.reshape(1, _HID), dinv)


def _tc_final(z, u, dinv, W, b, r, index, Wf, bf):
    def body(z_ref, u_ref, dinv_ref, w_ref, b_ref, r_ref, idx_ref,
             wf_ref, bf_ref, o_ref, pooled):
        i = pl.program_id(0)
        v = jnp.concatenate([z_ref[ch] + u_ref[ch] for ch in range(4)],
                            axis=1)
        h = jnp.dot(v, w_ref[...], preferred_element_type=jnp.float32)
        h = h * dinv_ref[...] + b_ref[...] + r_ref[...]
        seg = (idx_ref[...] == jax.lax.broadcasted_iota(
            jnp.int32, (1, _G), 1)).astype(jnp.float32)

        @pl.when(i == 0)
        def _():
            pooled[...] = jnp.zeros_like(pooled)

        pooled[...] += lax.dot_general(
            seg, h, (((0,), (0,)), ((), ())),
            preferred_element_type=jnp.float32)

        @pl.when(i == _NBLK - 1)
        def _():
            o_ref[...] = jnp.dot(
                pooled[...], wf_ref[...],
                preferred_element_type=jnp.float32) + bf_ref[...]

    return pl.pallas_call(
        body,
        grid=(_NBLK,),
        in_specs=[
            pl.BlockSpec((4, _RB, 128), lambda i: (0, i, 0)),
            pl.BlockSpec((4, _RB, 128), lambda i: (0, i, 0)),
            pl.BlockSpec((_RB, 1), lambda i: (i, 0)),
            pl.BlockSpec((_HID, _HID), lambda i: (0, 0)),
            pl.BlockSpec((1, _HID), lambda i: (0, 0)),
            pl.BlockSpec((_RB, _HID), lambda i: (i, 0)),
            pl.BlockSpec((_RB, 1), lambda i: (i, 0)),
            pl.BlockSpec((_HID, _OUT_C), lambda i: (0, 0)),
            pl.BlockSpec((1, _OUT_C), lambda i: (0, 0)),
        ],
        out_specs=pl.BlockSpec((_G, _OUT_C), lambda i: (0, 0)),
        out_shape=jax.ShapeDtypeStruct((_G, _OUT_C), jnp.float32),
        scratch_shapes=[pltpu.VMEM((_G, _HID), jnp.float32)],
    )(z, u, dinv, W, b.reshape(1, _HID), r, index.reshape(_N, 1),
      Wf, bf.reshape(1, _OUT_C))


def kernel(x, edge_index, index, W0, b0, W1, b1, g1, be1, W2, b2, g2, be2,
           W3, b3, g3, be3, Wf, bf):
    src = edge_index[0]
    dst = edge_index[1]
    pad = _EP - _E
    ar = jnp.arange(pad, dtype=jnp.int32)
    pad_src = (ar * 7919) % _N
    pad_dst = _N + (ar % (_NP - _N))
    srcp = jnp.concatenate([src, pad_src])
    dstp = jnp.concatenate([dst, pad_dst]).reshape(32, _NB, _B)
    dstp2 = dstp.reshape(16, _NBS, _BS)
    src_off4 = (srcp.reshape(1, 16, _NBS, _BS)
                + (jnp.arange(4, dtype=jnp.int32) * _N).reshape(4, 1, 1, 1))

    degp = _sc_degree(dstp)
    deg2 = degp[:, :_N].reshape(2, _N, 1)
    dinv, u0 = _tc_scale0(deg2, x)

    z0 = _sc_prop(u0.reshape(2 * _N, 128), src_off4[:2], dstp2, 2)
    h, st = _tc_combine(z0, u0, dinv, W0, b0, None)

    for (W, b, g, be) in ((W1, b1, g1, be1), (W2, b2, g2, be2)):
        u = _tc_bn_scale(h, st, g, be, dinv)
        z = _sc_prop(u.reshape(4 * _N, 128), src_off4, dstp2, 4)
        h, st = _tc_combine(z, u, dinv, W, b, h)

    u3 = _tc_bn_scale(h, st, g3, be3, dinv)
    z3 = _sc_prop(u3.reshape(4 * _N, 128), src_off4, dstp2, 4)
    return _tc_final(z3, u3, dinv, W3, b3, h, index, Wf, bf)
